# baseline (speedup 1.0000x reference)
import functools

import jax
import jax.numpy as jnp
from jax import lax
from jax.experimental import pallas as pl
from jax.experimental.pallas import tpu as pltpu
from jax.experimental.pallas import tpu_sc as plsc

N = 10000
E = 320000
D = 128
DH = D // 2
NP = 10240
RPT = NP // 16
CH = 128
G = 16
NG = 10
NCH = NG * G
EPAD = 16 * NCH * CH
DUMMY = N
NSLOT = 4

_MESH = plsc.VectorSubcoreMesh(core_axis_name="c", subcore_axis_name="s")


@functools.partial(
    pl.kernel,
    out_type=jax.ShapeDtypeStruct((32, RPT, 16), jnp.float32),
    mesh=_MESH,
    scratch_types=[
        pltpu.VMEM((NCH, CH), jnp.int32),
        pltpu.VMEM((CH, 16), jnp.float32),
        pltpu.VMEM_SHARED((NP, 16), jnp.float32),
    ],
)
def _deg_kernel(dstf_hbm, dstr_hbm, out_hbm, dst_vm, ones_vm, acc_sh):
    c = lax.axis_index("c")
    s = lax.axis_index("s")

    @pl.when(c == 0)
    def _():
        pltpu.sync_copy(dstf_hbm.at[pl.ds(s * NCH, NCH)], dst_vm)

    @pl.when(c == 1)
    def _():
        pltpu.sync_copy(dstr_hbm.at[pl.ds(s * NCH, NCH)], dst_vm)

    zero = jnp.zeros((16,), jnp.float32)

    @pl.loop(0, CH)
    def _(i):
        ones_vm[i, :] = zero

    @pl.loop(0, RPT // CH)
    def _(k):
        pltpu.sync_copy(ones_vm, acc_sh.at[pl.ds(s * RPT + k * CH, CH)])

    one = jnp.full((16,), 1.0, jnp.float32)

    @pl.loop(0, CH)
    def _(i):
        ones_vm[i, :] = one

    plsc.subcore_barrier()

    @pl.loop(0, NCH)
    def _(j):
        pltpu.sync_copy(ones_vm, acc_sh.at[dst_vm.at[j]], add=True)

    plsc.subcore_barrier()
    pltpu.sync_copy(acc_sh.at[pl.ds(s * RPT, RPT)], out_hbm.at[c * 16 + s])


@functools.partial(
    pl.kernel,
    out_type=[jax.ShapeDtypeStruct((32, RPT, DH), jnp.float32),
              jax.ShapeDtypeStruct((32, RPT, DH), jnp.float32)],
    mesh=_MESH,
    scratch_types=[
        pltpu.VMEM((G, CH), jnp.int32),
        pltpu.VMEM((G, CH), jnp.int32),
        [pltpu.VMEM((CH, DH), jnp.float32)] * NSLOT,
        pltpu.VMEM_SHARED((NP, DH), jnp.float32),
        pltpu.VMEM_SHARED((NP, DH), jnp.float32),
        [pltpu.SemaphoreType.DMA] * NSLOT,
        [pltpu.SemaphoreType.DMA] * NSLOT,
    ],
    compiler_params=pltpu.CompilerParams(use_tc_tiling_on_sc=False),
)
def _spmm_kernel(tf0_hbm, tf1_hbm, tr0_hbm, tr1_hbm,
                 srcf_hbm, dstf_hbm, srcr_hbm, dstr_hbm,
                 outf_hbm, outr_hbm,
                 src_vm, dst_vm, rows, accf_sh, accr_sh, gsem, ssem):
    c = lax.axis_index("c")
    s = lax.axis_index("s")

    zero = jnp.zeros((16,), jnp.float32)

    @pl.loop(0, CH)
    def _(i):
        for k in range(DH // 16):
            rows[0][i, pl.ds(k * 16, 16)] = zero

    @pl.loop(0, RPT // CH)
    def _(k):
        pltpu.sync_copy(rows[0], accf_sh.at[pl.ds(s * RPT + k * CH, CH)])
        pltpu.sync_copy(rows[0], accr_sh.at[pl.ds(s * RPT + k * CH, CH)])

    plsc.subcore_barrier()

    def run_dir(tbl, src_hbm, dst_hbm, acc):
        def drain_gather(m):
            pltpu.make_async_copy(tbl.at[pl.ds(0, CH)], rows[m],
                                  gsem[m]).wait()

        def drain_scatter(m):
            pltpu.make_async_copy(rows[m], acc.at[pl.ds(0, CH)],
                                  ssem[m]).wait()

        @pl.loop(0, NG)
        def _(g):
            base = s * NCH + g * G

            @pl.when(g > 0)
            def _():
                for m in range(NSLOT):
                    drain_scatter(m)

            pltpu.sync_copy(src_hbm.at[pl.ds(base, G)], src_vm)
            pltpu.sync_copy(dst_hbm.at[pl.ds(base, G)], dst_vm)

            @pl.loop(0, G // NSLOT)
            def _(r):
                q = r * NSLOT
                for m in range(NSLOT):
                    @pl.when(r > 0)
                    def _(m=m):
                        drain_scatter(m)
                    pltpu.async_copy(tbl.at[src_vm.at[q + m]], rows[m],
                                     gsem[m])
                for m in range(NSLOT):
                    drain_gather(m)
                    pltpu.async_copy(rows[m], acc.at[dst_vm.at[q + m]],
                                     ssem[m], add=True)

        for m in range(NSLOT):
            drain_scatter(m)

    @pl.when(c == 0)
    def _():
        run_dir(tf0_hbm, srcf_hbm, dstf_hbm, accf_sh)
        run_dir(tr0_hbm, srcr_hbm, dstr_hbm, accr_sh)

    @pl.when(c == 1)
    def _():
        run_dir(tf1_hbm, srcf_hbm, dstf_hbm, accf_sh)
        run_dir(tr1_hbm, srcr_hbm, dstr_hbm, accr_sh)

    plsc.subcore_barrier()
    pltpu.sync_copy(accf_sh.at[pl.ds(s * RPT, RPT)], outf_hbm.at[c * 16 + s])
    pltpu.sync_copy(accr_sh.at[pl.ds(s * RPT, RPT)], outr_hbm.at[c * 16 + s])


def _mm_t(a, w):
    return lax.dot_general(a, w, (((1,), (1,)), ((), ())),
                           preferred_element_type=jnp.float32)


def _dinv(deg_ref):
    return lax.rsqrt(deg_ref[:, 0:1] + 1.0)


def _tc_pre_body(x_ref, w1_ref, degf_ref, degr_ref,
                 hf0_ref, hf1_ref, hr0_ref, hr1_ref):
    h = jnp.dot(x_ref[...], w1_ref[...], preferred_element_type=jnp.float32)
    hf = h * _dinv(degf_ref)
    hr = h * _dinv(degr_ref)
    hf0_ref[...] = hf[:, :DH]
    hf1_ref[...] = hf[:, DH:]
    hr0_ref[...] = hr[:, :DH]
    hr1_ref[...] = hr[:, DH:]


def _cat(ref0, ref1):
    return jnp.concatenate([ref0[0], ref1[0]], axis=1)


def _tc_mid_h_body(accf0_ref, accf1_ref, accr0_ref, accr1_ref,
                   hf0_ref, hf1_ref, hr0_ref, hr1_ref, degf_ref, degr_ref,
                   w2_ref, w11_ref, w12_ref, b1_ref, bc1_ref,
                   of0_ref, of1_ref, or0_ref, or1_ref):
    dinvf = _dinv(degf_ref)
    dinvr = _dinv(degr_ref)
    hf = jnp.concatenate([hf0_ref[...], hf1_ref[...]], axis=1)
    hr = jnp.concatenate([hr0_ref[...], hr1_ref[...]], axis=1)
    c11 = jax.nn.relu((_cat(accf0_ref, accf1_ref) + hf) * dinvf
                      + bc1_ref[...])
    c12 = jax.nn.relu((_cat(accr0_ref, accr1_ref) + hr) * dinvr
                      + bc1_ref[...])
    g = jax.nn.sigmoid(_mm_t(c11, w11_ref[...]) + _mm_t(c12, w12_ref[...])
                       + b1_ref[...])
    hmid = g * c11 + (1.0 - g) * c12
    h2 = jnp.dot(hmid, w2_ref[...], preferred_element_type=jnp.float32)
    of = h2 * dinvf
    orr = h2 * dinvr
    of0_ref[...] = of[:, :DH]
    of1_ref[...] = of[:, DH:]
    or0_ref[...] = orr[:, :DH]
    or1_ref[...] = orr[:, DH:]


def _tc_fin_body(accf0_ref, accf1_ref, accr0_ref, accr1_ref,
                 hf0_ref, hf1_ref, hr0_ref, hr1_ref, degf_ref, degr_ref,
                 w21_ref, w22_ref, b2_ref, bc2_ref, out_ref):
    dinvf = _dinv(degf_ref)
    dinvr = _dinv(degr_ref)
    hf = jnp.concatenate([hf0_ref[...], hf1_ref[...]], axis=1)
    hr = jnp.concatenate([hr0_ref[...], hr1_ref[...]], axis=1)
    c21 = jax.nn.relu((_cat(accf0_ref, accf1_ref) + hf) * dinvf
                      + bc2_ref[...])
    c22 = jax.nn.relu((_cat(accr0_ref, accr1_ref) + hr) * dinvr
                      + bc2_ref[...])
    g2 = jax.nn.sigmoid(_mm_t(c21, w21_ref[...]) + _mm_t(c22, w22_ref[...])
                        + b2_ref[...])
    out_ref[...] = g2 * c21 + (1.0 - g2) * c22


def _row_spec(rb, d):
    return pl.BlockSpec((rb, d), lambda i: (i, 0))


def _full_spec(shape):
    nd = len(shape)
    return pl.BlockSpec(shape, lambda i: (0,) * nd)


def _acc_spec(rb, half):
    return pl.BlockSpec((1, rb, DH), lambda i, h=half: (h, i, 0))


def kernel(x, edge_index, edge_index_reverse, W1, bc1, W2, bc2,
           w11, w12, b1, w21, w22, b2):
    xp = jnp.pad(x, ((0, NP - N), (0, 0)))

    def pack(ei):
        src = jnp.concatenate([ei[0], jnp.zeros((EPAD - E,), ei.dtype)])
        dst = jnp.concatenate(
            [ei[1], jnp.full((EPAD - E,), DUMMY, ei.dtype)])
        return src.reshape(16 * NCH, CH), dst.reshape(16 * NCH, CH)

    srcf, dstf = pack(edge_index)
    srcr, dstr = pack(edge_index_reverse)
    b1r = b1.reshape(1, D)
    b2r = b2.reshape(1, D)
    bc1r = bc1.reshape(1, D)
    bc2r = bc2.reshape(1, D)

    deg = _deg_kernel(dstf, dstr).reshape(2, NP, 16)
    degf, degr = deg[0], deg[1]

    RB = 512
    h1f0, h1f1, h1r0, h1r1 = pl.pallas_call(
        _tc_pre_body,
        grid=(NP // RB,),
        in_specs=[_row_spec(RB, D), _full_spec((D, D)),
                  _row_spec(RB, 16), _row_spec(RB, 16)],
        out_specs=[_row_spec(RB, DH)] * 4,
        out_shape=[jax.ShapeDtypeStruct((NP, DH), jnp.float32)] * 4,
    )(xp, W1, degf, degr)

    accf, accr = _spmm_kernel(h1f0, h1f1, h1r0, h1r1, srcf, dstf, srcr, dstr)
    accf = accf.reshape(2, NP, DH)
    accr = accr.reshape(2, NP, DH)

    h2f0, h2f1, h2r0, h2r1 = pl.pallas_call(
        _tc_mid_h_body,
        grid=(NP // RB,),
        in_specs=[_acc_spec(RB, 0), _acc_spec(RB, 1),
                  _acc_spec(RB, 0), _acc_spec(RB, 1),
                  _row_spec(RB, DH), _row_spec(RB, DH),
                  _row_spec(RB, DH), _row_spec(RB, DH),
                  _row_spec(RB, 16), _row_spec(RB, 16),
                  _full_spec((D, D)), _full_spec((D, D)), _full_spec((D, D)),
                  _full_spec((1, D)), _full_spec((1, D))],
        out_specs=[_row_spec(RB, DH)] * 4,
        out_shape=[jax.ShapeDtypeStruct((NP, DH), jnp.float32)] * 4,
    )(accf, accf, accr, accr, h1f0, h1f1, h1r0, h1r1, degf, degr,
      W2, w11, w12, b1r, bc1r)

    acc2f, acc2r = _spmm_kernel(h2f0, h2f1, h2r0, h2r1, srcf, dstf,
                                srcr, dstr)
    acc2f = acc2f.reshape(2, NP, DH)
    acc2r = acc2r.reshape(2, NP, DH)

    RF = 400
    out = pl.pallas_call(
        _tc_fin_body,
        grid=(N // RF,),
        in_specs=[_acc_spec(RF, 0), _acc_spec(RF, 1),
                  _acc_spec(RF, 0), _acc_spec(RF, 1),
                  _row_spec(RF, DH), _row_spec(RF, DH),
                  _row_spec(RF, DH), _row_spec(RF, DH),
                  _row_spec(RF, 16), _row_spec(RF, 16),
                  _full_spec((D, D)), _full_spec((D, D)),
                  _full_spec((1, D)), _full_spec((1, D))],
        out_specs=_row_spec(RF, D),
        out_shape=jax.ShapeDtypeStruct((N, D), jnp.float32),
    )(acc2f, acc2f, acc2r, acc2r, h2f0, h2f1, h2r0, h2r1, degf, degr,
      w21, w22, b2r, bc2r)
    return out


# speedup vs baseline: 1.0274x; 1.0274x over previous
import functools

import jax
import jax.numpy as jnp
from jax import lax
from jax.experimental import pallas as pl
from jax.experimental.pallas import tpu as pltpu
from jax.experimental.pallas import tpu_sc as plsc

N = 10000
E = 320000
D = 128
NP = 10240
RPT = NP // 16
CH = 128
G = 16
NG = 10
NCH = NG * G
EPAD = 16 * NCH * CH
DUMMY = N

_MESH = plsc.VectorSubcoreMesh(core_axis_name="c", subcore_axis_name="s")


@functools.partial(
    pl.kernel,
    out_type=jax.ShapeDtypeStruct((32, RPT, 16), jnp.float32),
    mesh=_MESH,
    scratch_types=[
        pltpu.VMEM((NCH, CH), jnp.int32),
        pltpu.VMEM((CH, 16), jnp.float32),
        pltpu.VMEM_SHARED((NP, 16), jnp.float32),
    ],
)
def _deg_kernel(dstf_hbm, dstr_hbm, out_hbm, dst_vm, ones_vm, acc_sh):
    c = lax.axis_index("c")
    s = lax.axis_index("s")

    @pl.when(c == 0)
    def _():
        pltpu.sync_copy(dstf_hbm.at[pl.ds(s * NCH, NCH)], dst_vm)

    @pl.when(c == 1)
    def _():
        pltpu.sync_copy(dstr_hbm.at[pl.ds(s * NCH, NCH)], dst_vm)

    zero = jnp.zeros((16,), jnp.float32)

    @pl.loop(0, CH)
    def _(i):
        ones_vm[i, :] = zero

    @pl.loop(0, RPT // CH)
    def _(k):
        pltpu.sync_copy(ones_vm, acc_sh.at[pl.ds(s * RPT + k * CH, CH)])

    one = jnp.full((16,), 1.0, jnp.float32)

    @pl.loop(0, CH)
    def _(i):
        ones_vm[i, :] = one

    plsc.subcore_barrier()

    @pl.loop(0, NCH)
    def _(j):
        pltpu.sync_copy(ones_vm, acc_sh.at[dst_vm.at[j]], add=True)

    plsc.subcore_barrier()
    pltpu.sync_copy(acc_sh.at[pl.ds(s * RPT, RPT)], out_hbm.at[c * 16 + s])


@functools.partial(
    pl.kernel,
    out_type=jax.ShapeDtypeStruct((32, RPT, D), jnp.float32),
    mesh=_MESH,
    scratch_types=[
        pltpu.VMEM((G, CH), jnp.int32),
        pltpu.VMEM((G, CH), jnp.int32),
        pltpu.VMEM((CH, D), jnp.float32),
        pltpu.VMEM((CH, D), jnp.float32),
        pltpu.VMEM_SHARED((NP, D), jnp.float32),
        pltpu.SemaphoreType.DMA,
        pltpu.SemaphoreType.DMA,
    ],
)
def _spmm_kernel(tf_hbm, tr_hbm, srcf_hbm, dstf_hbm, srcr_hbm, dstr_hbm,
                 out_hbm, src_vm, dst_vm, rows_a, rows_b, acc_sh, sem_a,
                 sem_b):
    c = lax.axis_index("c")
    s = lax.axis_index("s")

    zero = jnp.zeros((16,), jnp.float32)

    @pl.loop(0, CH)
    def _(i):
        for k in range(D // 16):
            rows_a[i, pl.ds(k * 16, 16)] = zero

    @pl.loop(0, RPT // CH)
    def _(k):
        pltpu.sync_copy(rows_a, acc_sh.at[pl.ds(s * RPT + k * CH, CH)])

    plsc.subcore_barrier()

    def run_dir(tbl, src_hbm, dst_hbm):
        dummy = tbl.at[pl.ds(0, CH)]

        def wait_a():
            pltpu.make_async_copy(dummy, rows_a, sem_a).wait()

        def wait_b():
            pltpu.make_async_copy(dummy, rows_b, sem_b).wait()

        @pl.loop(0, NG)
        def _(g):
            base = s * NCH + g * G
            pltpu.sync_copy(src_hbm.at[pl.ds(base, G)], src_vm)
            pltpu.sync_copy(dst_hbm.at[pl.ds(base, G)], dst_vm)
            pltpu.async_copy(tbl.at[src_vm.at[0]], rows_a, sem_a)

            @pl.loop(0, G // 2 - 1)
            def _(k):
                pltpu.async_copy(tbl.at[src_vm.at[2 * k + 1]], rows_b, sem_b)
                wait_a()
                pltpu.sync_copy(rows_a, acc_sh.at[dst_vm.at[2 * k]], add=True)
                pltpu.async_copy(tbl.at[src_vm.at[2 * k + 2]], rows_a, sem_a)
                wait_b()
                pltpu.sync_copy(rows_b, acc_sh.at[dst_vm.at[2 * k + 1]],
                                add=True)

            pltpu.async_copy(tbl.at[src_vm.at[G - 1]], rows_b, sem_b)
            wait_a()
            pltpu.sync_copy(rows_a, acc_sh.at[dst_vm.at[G - 2]], add=True)
            wait_b()
            pltpu.sync_copy(rows_b, acc_sh.at[dst_vm.at[G - 1]], add=True)

    @pl.when(c == 0)
    def _():
        run_dir(tf_hbm, srcf_hbm, dstf_hbm)

    @pl.when(c == 1)
    def _():
        run_dir(tr_hbm, srcr_hbm, dstr_hbm)

    plsc.subcore_barrier()
    pltpu.sync_copy(acc_sh.at[pl.ds(s * RPT, RPT)], out_hbm.at[c * 16 + s])


def _mm_t(a, w):
    return lax.dot_general(a, w, (((1,), (1,)), ((), ())),
                           preferred_element_type=jnp.float32)


def _dinv(deg_ref):
    return lax.rsqrt(deg_ref[:, 0:1] + 1.0)


def _tc_mm_body(x_ref, w1_ref, h_ref):
    h_ref[...] = jnp.dot(x_ref[...], w1_ref[...],
                         preferred_element_type=jnp.float32)


def _tc_scale_body(h_ref, degf_ref, degr_ref, hf_ref, hr_ref):
    h = h_ref[...]
    hf_ref[...] = h * _dinv(degf_ref)
    hr_ref[...] = h * _dinv(degr_ref)


def _tc_mid_body(accf_ref, accr_ref, hf_ref, hr_ref, degf_ref, degr_ref,
                 w2_ref, w11_ref, w12_ref, b1_ref, bc1_ref, of_ref, or_ref):
    dinvf = _dinv(degf_ref)
    dinvr = _dinv(degr_ref)
    c11 = jax.nn.relu((accf_ref[0] + hf_ref[...]) * dinvf + bc1_ref[...])
    c12 = jax.nn.relu((accr_ref[0] + hr_ref[...]) * dinvr + bc1_ref[...])
    g = jax.nn.sigmoid(_mm_t(c11, w11_ref[...]) + _mm_t(c12, w12_ref[...])
                       + b1_ref[...])
    hmid = g * c11 + (1.0 - g) * c12
    h2 = jnp.dot(hmid, w2_ref[...], preferred_element_type=jnp.float32)
    of_ref[...] = h2 * dinvf
    or_ref[...] = h2 * dinvr


def _tc_fin_body(accf_ref, accr_ref, hf_ref, hr_ref, degf_ref, degr_ref,
                 w21_ref, w22_ref, b2_ref, bc2_ref, out_ref):
    dinvf = _dinv(degf_ref)
    dinvr = _dinv(degr_ref)
    c21 = jax.nn.relu((accf_ref[0] + hf_ref[...]) * dinvf + bc2_ref[...])
    c22 = jax.nn.relu((accr_ref[0] + hr_ref[...]) * dinvr + bc2_ref[...])
    g2 = jax.nn.sigmoid(_mm_t(c21, w21_ref[...]) + _mm_t(c22, w22_ref[...])
                        + b2_ref[...])
    out_ref[...] = g2 * c21 + (1.0 - g2) * c22


def _row_spec(rb, d):
    return pl.BlockSpec((rb, d), lambda i: (i, 0))


def _full_spec(shape):
    nd = len(shape)
    return pl.BlockSpec(shape, lambda i: (0,) * nd)


def _acc_spec(rb, half):
    return pl.BlockSpec((1, rb, D), lambda i, h=half: (h, i, 0))


def kernel(x, edge_index, edge_index_reverse, W1, bc1, W2, bc2,
           w11, w12, b1, w21, w22, b2):
    xp = jnp.pad(x, ((0, NP - N), (0, 0)))

    def pack(ei):
        src = jnp.concatenate([ei[0], jnp.zeros((EPAD - E,), ei.dtype)])
        dst = jnp.concatenate(
            [ei[1], jnp.full((EPAD - E,), DUMMY, ei.dtype)])
        return src.reshape(16 * NCH, CH), dst.reshape(16 * NCH, CH)

    srcf, dstf = pack(edge_index)
    srcr, dstr = pack(edge_index_reverse)
    b1r = b1.reshape(1, D)
    b2r = b2.reshape(1, D)
    bc1r = bc1.reshape(1, D)
    bc2r = bc2.reshape(1, D)

    RB = 512
    h1 = pl.pallas_call(
        _tc_mm_body,
        grid=(NP // RB,),
        in_specs=[_row_spec(RB, D), _full_spec((D, D))],
        out_specs=_row_spec(RB, D),
        out_shape=jax.ShapeDtypeStruct((NP, D), jnp.float32),
    )(xp, W1)

    deg = _deg_kernel(dstf, dstr).reshape(2, NP, 16)
    degf, degr = deg[0], deg[1]

    h1f, h1r = pl.pallas_call(
        _tc_scale_body,
        grid=(NP // RB,),
        in_specs=[_row_spec(RB, D), _row_spec(RB, 16), _row_spec(RB, 16)],
        out_specs=[_row_spec(RB, D)] * 2,
        out_shape=[jax.ShapeDtypeStruct((NP, D), jnp.float32)] * 2,
    )(h1, degf, degr)

    acc1 = _spmm_kernel(h1f, h1r, srcf, dstf, srcr, dstr).reshape(2, NP, D)

    h2f, h2r = pl.pallas_call(
        _tc_mid_body,
        grid=(NP // RB,),
        in_specs=[_acc_spec(RB, 0), _acc_spec(RB, 1),
                  _row_spec(RB, D), _row_spec(RB, D),
                  _row_spec(RB, 16), _row_spec(RB, 16),
                  _full_spec((D, D)), _full_spec((D, D)), _full_spec((D, D)),
                  _full_spec((1, D)), _full_spec((1, D))],
        out_specs=[_row_spec(RB, D)] * 2,
        out_shape=[jax.ShapeDtypeStruct((NP, D), jnp.float32)] * 2,
    )(acc1, acc1, h1f, h1r, degf, degr, W2, w11, w12, b1r, bc1r)

    acc2 = _spmm_kernel(h2f, h2r, srcf, dstf, srcr, dstr).reshape(2, NP, D)

    RF = 400
    out = pl.pallas_call(
        _tc_fin_body,
        grid=(N // RF,),
        in_specs=[_acc_spec(RF, 0), _acc_spec(RF, 1),
                  _row_spec(RF, D), _row_spec(RF, D),
                  _row_spec(RF, 16), _row_spec(RF, 16),
                  _full_spec((D, D)), _full_spec((D, D)),
                  _full_spec((1, D)), _full_spec((1, D))],
        out_specs=_row_spec(RF, D),
        out_shape=jax.ShapeDtypeStruct((N, D), jnp.float32),
    )(acc2, acc2, h2f, h2r, degf, degr, w21, w22, b2r, bc2r)
    return out


# speedup vs baseline: 2.5193x; 2.4521x over previous
import functools

import jax
import jax.numpy as jnp
from jax import lax
from jax.experimental import pallas as pl
from jax.experimental.pallas import tpu as pltpu
from jax.experimental.pallas import tpu_sc as plsc

N = 10000
E = 320000
D = 128
NP = 10240
RPT = NP // 16
CH = 128
G = 16
NG = 10
NCH = NG * G
EPAD = 16 * NCH * CH
DUMMY = N

_MESH = plsc.VectorSubcoreMesh(core_axis_name="c", subcore_axis_name="s")


@functools.partial(
    pl.kernel,
    out_type=jax.ShapeDtypeStruct((32, RPT, 16), jnp.float32),
    mesh=_MESH,
    scratch_types=[
        pltpu.VMEM((NCH, CH), jnp.int32),
        pltpu.VMEM((CH, 16), jnp.float32),
        pltpu.VMEM_SHARED((NP, 16), jnp.float32),
    ],
)
def _deg_kernel(dstf_hbm, dstr_hbm, out_hbm, dst_vm, ones_vm, acc_sh):
    c = lax.axis_index("c")
    s = lax.axis_index("s")

    @pl.when(c == 0)
    def _():
        pltpu.sync_copy(dstf_hbm.at[pl.ds(s * NCH, NCH)], dst_vm)

    @pl.when(c == 1)
    def _():
        pltpu.sync_copy(dstr_hbm.at[pl.ds(s * NCH, NCH)], dst_vm)

    zero = jnp.zeros((16,), jnp.float32)

    @pl.loop(0, CH)
    def _(i):
        ones_vm[i, :] = zero

    @pl.loop(0, RPT // CH)
    def _(k):
        pltpu.sync_copy(ones_vm, acc_sh.at[pl.ds(s * RPT + k * CH, CH)])

    one = jnp.full((16,), 1.0, jnp.float32)

    @pl.loop(0, CH)
    def _(i):
        ones_vm[i, :] = one

    plsc.subcore_barrier()

    @pl.loop(0, NCH)
    def _(j):
        pltpu.sync_copy(ones_vm, acc_sh.at[dst_vm.at[j]], add=True)

    plsc.subcore_barrier()
    pltpu.sync_copy(acc_sh.at[pl.ds(s * RPT, RPT)], out_hbm.at[c * 16 + s])


@functools.partial(
    pl.kernel,
    out_type=jax.ShapeDtypeStruct((32, RPT, D), jnp.float32),
    mesh=_MESH,
    scratch_types=[
        pltpu.VMEM((G, CH), jnp.int32),
        pltpu.VMEM((G, CH), jnp.int32),
        pltpu.VMEM((CH, D), jnp.float32),
        pltpu.VMEM((CH, D), jnp.float32),
        pltpu.VMEM_SHARED((NP, D), jnp.float32),
        pltpu.SemaphoreType.DMA,
        pltpu.SemaphoreType.DMA,
    ],
)
def _spmm_kernel(tf_hbm, tr_hbm, srcf_hbm, dstf_hbm, srcr_hbm, dstr_hbm,
                 out_hbm, src_vm, dst_vm, rows_a, rows_b, acc_sh, sem_a,
                 sem_b):
    c = lax.axis_index("c")
    s = lax.axis_index("s")

    zero = jnp.zeros((16,), jnp.float32)

    @pl.loop(0, CH)
    def _(i):
        for k in range(D // 16):
            rows_a[i, pl.ds(k * 16, 16)] = zero

    @pl.loop(0, RPT // CH)
    def _(k):
        pltpu.sync_copy(rows_a, acc_sh.at[pl.ds(s * RPT + k * CH, CH)])

    plsc.subcore_barrier()

    def run_dir(tbl, src_hbm, dst_hbm):
        dummy = tbl.at[pl.ds(0, CH)]

        def wait_a():
            pltpu.make_async_copy(dummy, rows_a, sem_a).wait()

        def wait_b():
            pltpu.make_async_copy(dummy, rows_b, sem_b).wait()

        @pl.loop(0, NG)
        def _(g):
            base = s * NCH + g * G
            pltpu.sync_copy(src_hbm.at[pl.ds(base, G)], src_vm)
            pltpu.sync_copy(dst_hbm.at[pl.ds(base, G)], dst_vm)
            pltpu.async_copy(tbl.at[src_vm.at[0]], rows_a, sem_a)

            @pl.loop(0, G // 2 - 1)
            def _(k):
                pltpu.async_copy(tbl.at[src_vm.at[2 * k + 1]], rows_b, sem_b)
                wait_a()
                pltpu.sync_copy(rows_a, acc_sh.at[dst_vm.at[2 * k]], add=True)
                pltpu.async_copy(tbl.at[src_vm.at[2 * k + 2]], rows_a, sem_a)
                wait_b()
                pltpu.sync_copy(rows_b, acc_sh.at[dst_vm.at[2 * k + 1]],
                                add=True)

            pltpu.async_copy(tbl.at[src_vm.at[G - 1]], rows_b, sem_b)
            wait_a()
            pltpu.sync_copy(rows_a, acc_sh.at[dst_vm.at[G - 2]], add=True)
            wait_b()
            pltpu.sync_copy(rows_b, acc_sh.at[dst_vm.at[G - 1]], add=True)

    @pl.when(c == 0)
    def _():
        run_dir(tf_hbm, srcf_hbm, dstf_hbm)

    @pl.when(c == 1)
    def _():
        run_dir(tr_hbm, srcr_hbm, dstr_hbm)

    plsc.subcore_barrier()
    pltpu.sync_copy(acc_sh.at[pl.ds(s * RPT, RPT)], out_hbm.at[c * 16 + s])


def _mm_t(a, w):
    return lax.dot_general(a, w, (((1,), (1,)), ((), ())),
                           preferred_element_type=jnp.float32)


def _dinv(deg_ref):
    return lax.rsqrt(deg_ref[:, 0:1] + 1.0)


def _tc_pre_body(x_ref, w1_ref, degf_ref, degr_ref, hf_ref, hr_ref):
    h = jnp.dot(x_ref[...], w1_ref[...], preferred_element_type=jnp.float32)
    hf_ref[...] = h * _dinv(degf_ref)
    hr_ref[...] = h * _dinv(degr_ref)


def _tc_mid_body(accf_ref, accr_ref, hf_ref, hr_ref, degf_ref, degr_ref,
                 w2_ref, w11_ref, w12_ref, b1_ref, bc1_ref, of_ref, or_ref):
    dinvf = _dinv(degf_ref)
    dinvr = _dinv(degr_ref)
    c11 = jax.nn.relu((accf_ref[0] + hf_ref[...]) * dinvf + bc1_ref[...])
    c12 = jax.nn.relu((accr_ref[0] + hr_ref[...]) * dinvr + bc1_ref[...])
    g = jax.nn.sigmoid(_mm_t(c11, w11_ref[...]) + _mm_t(c12, w12_ref[...])
                       + b1_ref[...])
    hmid = g * c11 + (1.0 - g) * c12
    h2 = jnp.dot(hmid, w2_ref[...], preferred_element_type=jnp.float32)
    of_ref[...] = h2 * dinvf
    or_ref[...] = h2 * dinvr


def _tc_fin_body(accf_ref, accr_ref, hf_ref, hr_ref, degf_ref, degr_ref,
                 w21_ref, w22_ref, b2_ref, bc2_ref, out_ref):
    dinvf = _dinv(degf_ref)
    dinvr = _dinv(degr_ref)
    c21 = jax.nn.relu((accf_ref[0] + hf_ref[...]) * dinvf + bc2_ref[...])
    c22 = jax.nn.relu((accr_ref[0] + hr_ref[...]) * dinvr + bc2_ref[...])
    g2 = jax.nn.sigmoid(_mm_t(c21, w21_ref[...]) + _mm_t(c22, w22_ref[...])
                        + b2_ref[...])
    out_ref[...] = g2 * c21 + (1.0 - g2) * c22


def _row_spec(rb, d):
    return pl.BlockSpec((rb, d), lambda i: (i, 0))


def _full_spec(shape):
    nd = len(shape)
    return pl.BlockSpec(shape, lambda i: (0,) * nd)


def _acc_spec(rb, half):
    return pl.BlockSpec((1, rb, D), lambda i, h=half: (h, i, 0))


def kernel(x, edge_index, edge_index_reverse, W1, bc1, W2, bc2,
           w11, w12, b1, w21, w22, b2):
    xp = jnp.pad(x, ((0, NP - N), (0, 0)))

    pad_src = (jnp.arange(EPAD - E, dtype=edge_index.dtype) * 7) % N
    pad_dst = N + (jnp.arange(EPAD - E, dtype=edge_index.dtype) % (NP - N))

    def pack(ei):
        src = jnp.concatenate([ei[0], pad_src])
        dst = jnp.concatenate([ei[1], pad_dst])
        return src.reshape(16 * NCH, CH), dst.reshape(16 * NCH, CH)

    srcf, dstf = pack(edge_index)
    srcr, dstr = pack(edge_index_reverse)
    b1r = b1.reshape(1, D)
    b2r = b2.reshape(1, D)
    bc1r = bc1.reshape(1, D)
    bc2r = bc2.reshape(1, D)

    deg = _deg_kernel(dstf, dstr).reshape(2, NP, 16)
    degf, degr = deg[0], deg[1]

    RB = 512
    h1f, h1r = pl.pallas_call(
        _tc_pre_body,
        grid=(NP // RB,),
        in_specs=[_row_spec(RB, D), _full_spec((D, D)),
                  _row_spec(RB, 16), _row_spec(RB, 16)],
        out_specs=[_row_spec(RB, D)] * 2,
        out_shape=[jax.ShapeDtypeStruct((NP, D), jnp.float32)] * 2,
    )(xp, W1, degf, degr)

    acc1 = _spmm_kernel(h1f, h1r, srcf, dstf, srcr, dstr).reshape(2, NP, D)

    h2f, h2r = pl.pallas_call(
        _tc_mid_body,
        grid=(NP // RB,),
        in_specs=[_acc_spec(RB, 0), _acc_spec(RB, 1),
                  _row_spec(RB, D), _row_spec(RB, D),
                  _row_spec(RB, 16), _row_spec(RB, 16),
                  _full_spec((D, D)), _full_spec((D, D)), _full_spec((D, D)),
                  _full_spec((1, D)), _full_spec((1, D))],
        out_specs=[_row_spec(RB, D)] * 2,
        out_shape=[jax.ShapeDtypeStruct((NP, D), jnp.float32)] * 2,
    )(acc1, acc1, h1f, h1r, degf, degr, W2, w11, w12, b1r, bc1r)

    acc2 = _spmm_kernel(h2f, h2r, srcf, dstf, srcr, dstr).reshape(2, NP, D)

    RF = 400
    out = pl.pallas_call(
        _tc_fin_body,
        grid=(N // RF,),
        in_specs=[_acc_spec(RF, 0), _acc_spec(RF, 1),
                  _row_spec(RF, D), _row_spec(RF, D),
                  _row_spec(RF, 16), _row_spec(RF, 16),
                  _full_spec((D, D)), _full_spec((D, D)),
                  _full_spec((1, D)), _full_spec((1, D))],
        out_specs=_row_spec(RF, D),
        out_shape=jax.ShapeDtypeStruct((N, D), jnp.float32),
    )(acc2, acc2, h2f, h2r, degf, degr, w21, w22, b2r, bc2r)
    return out
